# baseline (speedup 1.0000x reference)
import functools

import jax
import jax.numpy as jnp
from jax import lax
from jax.experimental import pallas as pl
from jax.experimental.pallas import tpu as pltpu
from jax.experimental.pallas import tpu_sc as plsc

_ROWS = 64
_DIM = 128
_BATCH = 16384
_NC, _NS = 2, 16
_NW = _NC * _NS
_BPW = _BATCH // _NW
_CHUNK = 128
_NCHUNK = _BPW // _CHUNK


def _normalize_body(tab_ref, out_ref):
    t = tab_ref[...]
    ssq = jnp.sum(t * t, axis=1, keepdims=True)
    out_ref[...] = t * lax.rsqrt(jnp.maximum(ssq, 1e-24))


_normalize = pl.pallas_call(
    _normalize_body,
    out_shape=jax.ShapeDtypeStruct((_ROWS, _DIM), jnp.float32),
)

_mesh = plsc.VectorSubcoreMesh(
    core_axis_name="c", subcore_axis_name="s", num_cores=_NC, num_subcores=_NS
)


@functools.partial(
    pl.kernel,
    out_type=jax.ShapeDtypeStruct((_BATCH, _DIM), jnp.float32),
    mesh=_mesh,
    scratch_types=[
        pltpu.VMEM((_NCHUNK, _CHUNK), jnp.int32),
        pltpu.VMEM((_NCHUNK, _CHUNK, _DIM), jnp.float32),
        pltpu.VMEM_SHARED((_ROWS, _DIM), jnp.float32),
        pltpu.SemaphoreType.DMA,
        pltpu.SemaphoreType.DMA,
        [pltpu.SemaphoreType.DMA] * _NCHUNK,
        pltpu.SemaphoreType.DMA,
    ],
)
def _gather(ntab_hbm, idx_hbm, out_hbm, idx_v, rows_v, stab, isem, tsem,
            gsems, ssem):
    sid = lax.axis_index("s")
    wid = sid * _NC + lax.axis_index("c")
    base = wid * _BPW

    _RPT = _ROWS // _NS
    icp = pltpu.async_copy(idx_hbm.at[wid], idx_v, isem)
    tcp = pltpu.async_copy(
        ntab_hbm.at[pl.ds(sid * _RPT, _RPT)],
        stab.at[pl.ds(sid * _RPT, _RPT)],
        tsem,
    )
    tcp.wait()
    plsc.subcore_barrier()
    icp.wait()

    gcps = [
        pltpu.async_copy(
            (ntab_hbm if g == 0 else stab).at[idx_v.at[g]], rows_v.at[g],
            gsems[g])
        for g in range(_NCHUNK)
    ]
    scps = []
    for g in range(_NCHUNK):
        gcps[g].wait()
        scps.append(
            pltpu.async_copy(
                rows_v.at[g], out_hbm.at[pl.ds(base + g * _CHUNK, _CHUNK)], ssem
            )
        )
    for c in scps:
        c.wait()


def kernel(indices, table):
    ntab = _normalize(table)
    idx3 = indices.astype(jnp.int32).reshape(_NW, _NCHUNK, _CHUNK)
    return _gather(ntab, idx3)


# speedup vs baseline: 1.1565x; 1.1565x over previous
import functools

import jax
import jax.numpy as jnp
from jax import lax
from jax.experimental import pallas as pl
from jax.experimental.pallas import tpu as pltpu
from jax.experimental.pallas import tpu_sc as plsc

_ROWS = 64
_DIM = 128
_BATCH = 16384
_NC, _NS = 2, 16
_NW = _NC * _NS
_BPW = _BATCH // _NW
_CHUNK = 128
_NCHUNK = _BPW // _CHUNK


def _normalize_body(tab_ref, out_ref):
    t = tab_ref[...]
    ssq = jnp.sum(t * t, axis=1, keepdims=True)
    out_ref[...] = t * lax.rsqrt(jnp.maximum(ssq, 1e-24))


_normalize = pl.pallas_call(
    _normalize_body,
    out_shape=jax.ShapeDtypeStruct((_ROWS, _DIM), jnp.float32),
)

_mesh = plsc.VectorSubcoreMesh(
    core_axis_name="c", subcore_axis_name="s", num_cores=_NC, num_subcores=_NS
)


@functools.partial(
    pl.kernel,
    out_type=jax.ShapeDtypeStruct((_BATCH, _DIM), jnp.float32),
    mesh=_mesh,
    scratch_types=[
        pltpu.VMEM((_NCHUNK, _CHUNK), jnp.int32),
        pltpu.VMEM((_NCHUNK, _CHUNK, _DIM), jnp.float32),
        pltpu.VMEM_SHARED((_ROWS, _DIM), jnp.float32),
        pltpu.SemaphoreType.DMA,
        pltpu.SemaphoreType.DMA,
        [pltpu.SemaphoreType.DMA] * _NCHUNK,
        pltpu.SemaphoreType.DMA,
    ],
)
def _gather(ntab_hbm, idx_hbm, out_hbm, idx_v, rows_v, stab, isem, tsem,
            gsems, ssem):
    sid = lax.axis_index("s")
    wid = sid * _NC + lax.axis_index("c")
    base = wid * _BPW

    _RPT = _ROWS // _NS
    icp = pltpu.async_copy(idx_hbm.at[wid], idx_v, isem)
    tcp = pltpu.async_copy(
        ntab_hbm.at[pl.ds(sid * _RPT, _RPT)],
        stab.at[pl.ds(sid * _RPT, _RPT)],
        tsem,
    )
    tcp.wait()
    plsc.subcore_barrier()
    icp.wait()

    gcps = [
        pltpu.async_copy(stab.at[idx_v.at[g]], rows_v.at[g], gsems[g])
        for g in range(_NCHUNK)
    ]
    scps = []
    for g in range(_NCHUNK):
        gcps[g].wait()
        scps.append(
            pltpu.async_copy(
                rows_v.at[g], out_hbm.at[pl.ds(base + g * _CHUNK, _CHUNK)], ssem
            )
        )
    for c in scps:
        c.wait()


def kernel(indices, table):
    ntab = _normalize(table)
    idx3 = indices.astype(jnp.int32).reshape(_NW, _NCHUNK, _CHUNK)
    return _gather(ntab, idx3)
